# baseline (speedup 1.0000x reference)
import functools

import jax
import jax.numpy as jnp
from jax import lax
from jax.experimental import pallas as pl
from jax.experimental.pallas import tpu as pltpu
from jax.experimental.pallas import tpu_sc as plsc

B = 16384
DEG = 32
D = 128
HD = D // 2
N_ROWS = 100000
NC = 2
NS = 16
NW = NC * NS
BPW = B // NW
CH = 128
NPC = CH // DEG
NCHUNK = BPW * DEG // CH
CVB = 2000


def _pack_table(table):
    def body(x_ref, o_ref):
        x = x_ref[...]
        lo = lax.bitcast_convert_type(
            x[:, :HD].astype(jnp.bfloat16), jnp.uint16).astype(jnp.int32)
        hi = lax.bitcast_convert_type(
            x[:, HD:].astype(jnp.bfloat16), jnp.uint16).astype(jnp.int32)
        o_ref[...] = jnp.bitwise_or(lax.shift_left(hi, 16), lo)

    return pl.pallas_call(
        body,
        grid=(N_ROWS // CVB,),
        in_specs=[pl.BlockSpec((CVB, D), lambda i: (i, 0))],
        out_specs=pl.BlockSpec((CVB, HD), lambda i: (i, 0)),
        out_shape=jax.ShapeDtypeStruct((N_ROWS, HD), jnp.int32),
    )(table)


def _sc_gather(table_i32, neigh_flat, nodes):
    mesh = plsc.VectorSubcoreMesh(core_axis_name="c", subcore_axis_name="s")

    @functools.partial(
        pl.kernel,
        mesh=mesh,
        compiler_params=pltpu.CompilerParams(use_tc_tiling_on_sc=False,
                                             needs_layout_passes=False),
        out_type=[
            jax.ShapeDtypeStruct((B, HD), jnp.int32),
            jax.ShapeDtypeStruct((B, D), jnp.float32),
        ],
        scratch_types=[
            pltpu.VMEM((BPW * DEG,), jnp.int32),
            pltpu.VMEM((BPW,), jnp.int32),
            pltpu.VMEM((2, CH, HD), jnp.int32),
            pltpu.VMEM((BPW, D), jnp.float32),
            pltpu.SemaphoreType.DMA((2,)),
        ],
    )
    def k(table_h, gidx_h, nidx_h, self_o, nsum_o, gidx, nidx, bufs, acc,
          gsem):
        c = lax.axis_index("c")
        s = lax.axis_index("s")
        base = (c * NS + s) * BPW

        pltpu.sync_copy(gidx_h.at[pl.ds(base * DEG, BPW * DEG)], gidx)
        pltpu.sync_copy(nidx_h.at[pl.ds(base, BPW)], nidx)

        def gcopy(ci, b):
            off = pl.multiple_of(ci * CH, CH)
            return pltpu.make_async_copy(
                table_h.at[gidx.at[pl.ds(off, CH)]], bufs.at[b], gsem.at[b])

        def reduce_chunk(ci, b):
            def nbody(n, carry):
                lo_acc = [jnp.zeros((16,), jnp.float32) for _ in range(4)]
                hi_acc = [jnp.zeros((16,), jnp.float32) for _ in range(4)]
                for r in range(DEG):
                    q = n * DEG + r
                    for g in range(4):
                        x = bufs[b, q, pl.ds(g * 16, 16)]
                        lo_acc[g] = lo_acc[g] + plsc.bitcast(
                            lax.shift_left(x, 16), jnp.float32)
                        hi_acc[g] = hi_acc[g] + plsc.bitcast(
                            jnp.bitwise_and(x, jnp.int32(-65536)),
                            jnp.float32)
                row = ci * NPC + n
                for g in range(4):
                    acc[row, pl.ds(g * 16, 16)] = lo_acc[g]
                    acc[row, pl.ds(HD + g * 16, 16)] = hi_acc[g]
                return carry

            lax.fori_loop(0, NPC, nbody, 0)

        gcopy(0, 0).start()

        def body(i, carry):
            c0 = 2 * i
            gcopy(c0 + 1, 1).start()
            gcopy(c0, 0).wait()
            reduce_chunk(c0, 0)

            @pl.when(i < NCHUNK // 2 - 1)
            def _():
                gcopy(c0 + 2, 0).start()

            gcopy(c0 + 1, 1).wait()
            reduce_chunk(c0 + 1, 1)
            return carry

        lax.fori_loop(0, NCHUNK // 2, body, 0)

        def sget(kk, b):
            return pltpu.make_async_copy(
                table_h.at[nidx.at[pl.ds(kk * CH, CH)]], bufs.at[b],
                gsem.at[b])

        for kk in range(0, BPW // CH, 2):
            sget(kk, 0).start()
            sget(kk + 1, 1).start()
            for b in range(2):
                sget(kk + b, b).wait()
                dst = pl.multiple_of(base + (kk + b) * CH, CH)
                pltpu.sync_copy(bufs.at[b], self_o.at[pl.ds(dst, CH)])

        pltpu.sync_copy(acc, nsum_o.at[pl.ds(pl.multiple_of(base, CH), BPW)])

    return k(table_i32, neigh_flat, nodes)


def _tc_dense(self_i32, nsum, W1, b1, gamma, beta):
    def body(x_ref, n_ref, w_ref, b_ref, g_ref, bb_ref, o_ref):
        xi = x_ref[...]
        lo = lax.bitcast_convert_type(
            jnp.bitwise_and(xi, jnp.int32(0xFFFF)).astype(jnp.uint16),
            jnp.bfloat16).astype(jnp.float32)
        hi = lax.bitcast_convert_type(
            lax.shift_right_logical(xi, 16).astype(jnp.uint16),
            jnp.bfloat16).astype(jnp.float32)
        n = n_ref[...] * (1.0 / DEG)
        w = w_ref[...]
        dn = (((1,), (1,)), ((), ()))
        lin = lax.dot_general(lo, w[:, :HD], dn,
                              preferred_element_type=jnp.float32)
        lin = lin + lax.dot_general(hi, w[:, HD:D], dn,
                                    preferred_element_type=jnp.float32)
        lin = lin + lax.dot_general(n, w[:, D:], dn,
                                    preferred_element_type=jnp.float32)
        lin = lin + b_ref[...]
        mu = jnp.mean(lin, axis=0, keepdims=True)
        xc = lin - mu
        var = jnp.mean(xc * xc, axis=0, keepdims=True)
        y = xc * lax.rsqrt(var + 1e-5) * g_ref[...] + bb_ref[...]
        o_ref[...] = jnp.maximum(y, 0.0)

    return pl.pallas_call(
        body,
        out_shape=jax.ShapeDtypeStruct((B, D), jnp.float32),
    )(self_i32, nsum, W1,
      b1.reshape(1, D), gamma.reshape(1, D), beta.reshape(1, D))


def kernel(nodes, neighbors, emb_table, W1, b1, gamma, beta):
    table_i32 = _pack_table(emb_table)
    self_i32, nsum = _sc_gather(table_i32, neighbors.reshape(-1), nodes)
    return _tc_dense(self_i32, nsum, W1, b1, gamma, beta)


# speedup vs baseline: 1.0020x; 1.0020x over previous
import functools

import jax
import jax.numpy as jnp
from jax import lax
from jax.experimental import pallas as pl
from jax.experimental.pallas import tpu as pltpu
from jax.experimental.pallas import tpu_sc as plsc

B = 16384
DEG = 32
D = 128
HD = D // 2
N_ROWS = 100000
NC = 2
NS = 16
NW = NC * NS
BPW = B // NW
CH = 128
NPC = CH // DEG
NCHUNK = BPW * DEG // CH
CVB = 2000


def _pack_table(table):
    def body(x_ref, o_ref):
        def rnd(v):
            bits = lax.bitcast_convert_type(v, jnp.int32)
            odd = jnp.bitwise_and(lax.shift_right_logical(bits, 16), 1)
            return lax.shift_right_logical(bits + 0x7FFF + odd, 16)

        x = x_ref[...]
        lo = rnd(x[:, :HD])
        hi = rnd(x[:, HD:])
        o_ref[...] = jnp.bitwise_or(lax.shift_left(hi, 16), lo)

    return pl.pallas_call(
        body,
        grid=(N_ROWS // CVB,),
        in_specs=[pl.BlockSpec((CVB, D), lambda i: (i, 0))],
        out_specs=pl.BlockSpec((CVB, HD), lambda i: (i, 0)),
        out_shape=jax.ShapeDtypeStruct((N_ROWS, HD), jnp.int32),
    )(table)


def _sc_gather(table_i32, neigh_flat, nodes):
    mesh = plsc.VectorSubcoreMesh(core_axis_name="c", subcore_axis_name="s")

    @functools.partial(
        pl.kernel,
        mesh=mesh,
        compiler_params=pltpu.CompilerParams(use_tc_tiling_on_sc=False,
                                             needs_layout_passes=False),
        out_type=[
            jax.ShapeDtypeStruct((B, HD), jnp.int32),
            jax.ShapeDtypeStruct((B, D), jnp.float32),
        ],
        scratch_types=[
            pltpu.VMEM((BPW * DEG,), jnp.int32),
            pltpu.VMEM((BPW,), jnp.int32),
            pltpu.VMEM((2, CH, HD), jnp.int32),
            pltpu.VMEM((BPW, D), jnp.float32),
            pltpu.SemaphoreType.DMA((2,)),
        ],
    )
    def k(table_h, gidx_h, nidx_h, self_o, nsum_o, gidx, nidx, bufs, acc,
          gsem):
        c = lax.axis_index("c")
        s = lax.axis_index("s")
        base = (c * NS + s) * BPW

        pltpu.sync_copy(gidx_h.at[pl.ds(base * DEG, BPW * DEG)], gidx)
        pltpu.sync_copy(nidx_h.at[pl.ds(base, BPW)], nidx)

        def gcopy(ci, b):
            off = pl.multiple_of(ci * CH, CH)
            return pltpu.make_async_copy(
                table_h.at[gidx.at[pl.ds(off, CH)]], bufs.at[b], gsem.at[b])

        def reduce_chunk(ci, b):
            def nbody(n, carry):
                lo_acc = [jnp.zeros((16,), jnp.float32) for _ in range(4)]
                hi_acc = [jnp.zeros((16,), jnp.float32) for _ in range(4)]
                for r in range(DEG):
                    q = n * DEG + r
                    for g in range(4):
                        x = bufs[b, q, pl.ds(g * 16, 16)]
                        lo_acc[g] = lo_acc[g] + plsc.bitcast(
                            lax.shift_left(x, 16), jnp.float32)
                        hi_acc[g] = hi_acc[g] + plsc.bitcast(
                            jnp.bitwise_and(x, jnp.int32(-65536)),
                            jnp.float32)
                row = ci * NPC + n
                for g in range(4):
                    acc[row, pl.ds(g * 16, 16)] = lo_acc[g]
                    acc[row, pl.ds(HD + g * 16, 16)] = hi_acc[g]
                return carry

            lax.fori_loop(0, NPC, nbody, 0)

        gcopy(0, 0).start()

        def body(i, carry):
            c0 = 2 * i
            gcopy(c0 + 1, 1).start()
            gcopy(c0, 0).wait()
            reduce_chunk(c0, 0)

            @pl.when(i < NCHUNK // 2 - 1)
            def _():
                gcopy(c0 + 2, 0).start()

            gcopy(c0 + 1, 1).wait()
            reduce_chunk(c0 + 1, 1)
            return carry

        lax.fori_loop(0, NCHUNK // 2, body, 0)

        def sget(kk, b):
            return pltpu.make_async_copy(
                table_h.at[nidx.at[pl.ds(kk * CH, CH)]], bufs.at[b],
                gsem.at[b])

        for kk in range(0, BPW // CH, 2):
            sget(kk, 0).start()
            sget(kk + 1, 1).start()
            for b in range(2):
                sget(kk + b, b).wait()
                dst = pl.multiple_of(base + (kk + b) * CH, CH)
                pltpu.sync_copy(bufs.at[b], self_o.at[pl.ds(dst, CH)])

        pltpu.sync_copy(acc, nsum_o.at[pl.ds(pl.multiple_of(base, CH), BPW)])

    return k(table_i32, neigh_flat, nodes)


def _tc_dense(self_i32, nsum, W1, b1, gamma, beta):
    def body(x_ref, n_ref, w_ref, b_ref, g_ref, bb_ref, o_ref):
        xi = x_ref[...]
        lo = lax.bitcast_convert_type(
            jnp.bitwise_and(xi, jnp.int32(0xFFFF)).astype(jnp.uint16),
            jnp.bfloat16).astype(jnp.float32)
        hi = lax.bitcast_convert_type(
            lax.shift_right_logical(xi, 16).astype(jnp.uint16),
            jnp.bfloat16).astype(jnp.float32)
        n = n_ref[...] * (1.0 / DEG)
        w = w_ref[...]
        dn = (((1,), (1,)), ((), ()))
        lin = lax.dot_general(lo, w[:, :HD], dn,
                              preferred_element_type=jnp.float32)
        lin = lin + lax.dot_general(hi, w[:, HD:D], dn,
                                    preferred_element_type=jnp.float32)
        lin = lin + lax.dot_general(n, w[:, D:], dn,
                                    preferred_element_type=jnp.float32)
        lin = lin + b_ref[...]
        mu = jnp.mean(lin, axis=0, keepdims=True)
        xc = lin - mu
        var = jnp.mean(xc * xc, axis=0, keepdims=True)
        y = xc * lax.rsqrt(var + 1e-5) * g_ref[...] + bb_ref[...]
        o_ref[...] = jnp.maximum(y, 0.0)

    return pl.pallas_call(
        body,
        out_shape=jax.ShapeDtypeStruct((B, D), jnp.float32),
    )(self_i32, nsum, W1,
      b1.reshape(1, D), gamma.reshape(1, D), beta.reshape(1, D))


def kernel(nodes, neighbors, emb_table, W1, b1, gamma, beta):
    table_i32 = _pack_table(emb_table)
    self_i32, nsum = _sc_gather(table_i32, neighbors.reshape(-1), nodes)
    return _tc_dense(self_i32, nsum, W1, b1, gamma, beta)


# speedup vs baseline: 1.1388x; 1.1366x over previous
import functools

import jax
import jax.numpy as jnp
from jax import lax
from jax.experimental import pallas as pl
from jax.experimental.pallas import tpu as pltpu
from jax.experimental.pallas import tpu_sc as plsc

B = 16384
DEG = 32
D = 128
HD = D // 2
N_ROWS = 100000
NC = 2
NS = 16
NW = NC * NS
BPW = B // NW
CH = 128
NPC = CH // DEG
NCHUNK = BPW * DEG // CH
TCB = 1024
NTB = B // TCB


def _sc_gather(table, neigh_flat, nodes):
    mesh = plsc.VectorSubcoreMesh(core_axis_name="c", subcore_axis_name="s")

    @functools.partial(
        pl.kernel,
        mesh=mesh,
        compiler_params=pltpu.CompilerParams(use_tc_tiling_on_sc=False,
                                             needs_layout_passes=False),
        out_type=[
            jax.ShapeDtypeStruct((B, D), jnp.float32),
            jax.ShapeDtypeStruct((B, D), jnp.float32),
        ],
        scratch_types=[
            pltpu.VMEM((BPW * DEG,), jnp.int32),
            pltpu.VMEM((BPW,), jnp.int32),
            pltpu.VMEM((2, CH, D), jnp.float32),
            pltpu.VMEM((BPW, D), jnp.float32),
            pltpu.SemaphoreType.DMA((2,)),
        ],
    )
    def k(table_h, gidx_h, nidx_h, self_o, nsum_o, gidx, nidx, bufs, acc,
          gsem):
        c = lax.axis_index("c")
        s = lax.axis_index("s")
        base = (c * NS + s) * BPW

        pltpu.sync_copy(gidx_h.at[pl.ds(base * DEG, BPW * DEG)], gidx)
        pltpu.sync_copy(nidx_h.at[pl.ds(base, BPW)], nidx)

        def gcopy(ci, b):
            off = pl.multiple_of(ci * CH, CH)
            return pltpu.make_async_copy(
                table_h.at[gidx.at[pl.ds(off, CH)]], bufs.at[b], gsem.at[b])

        def reduce_chunk(ci, b):
            def nbody(n, carry):
                sums = [jnp.zeros((16,), jnp.float32) for _ in range(8)]
                for r in range(DEG):
                    q = n * DEG + r
                    for g in range(8):
                        sums[g] = sums[g] + bufs[b, q, pl.ds(g * 16, 16)]
                row = ci * NPC + n
                for g in range(8):
                    acc[row, pl.ds(g * 16, 16)] = sums[g]
                return carry

            lax.fori_loop(0, NPC, nbody, 0)

        gcopy(0, 0).start()

        def body(i, carry):
            c0 = 2 * i
            gcopy(c0 + 1, 1).start()
            gcopy(c0, 0).wait()
            reduce_chunk(c0, 0)

            @pl.when(i < NCHUNK // 2 - 1)
            def _():
                gcopy(c0 + 2, 0).start()

            gcopy(c0 + 1, 1).wait()
            reduce_chunk(c0 + 1, 1)
            return carry

        lax.fori_loop(0, NCHUNK // 2, body, 0)

        def sget(kk, b):
            return pltpu.make_async_copy(
                table_h.at[nidx.at[pl.ds(kk * CH, CH)]], bufs.at[b],
                gsem.at[b])

        for kk in range(0, BPW // CH, 2):
            sget(kk, 0).start()
            sget(kk + 1, 1).start()
            for b in range(2):
                sget(kk + b, b).wait()
                dst = pl.multiple_of(base + (kk + b) * CH, CH)
                pltpu.sync_copy(bufs.at[b], self_o.at[pl.ds(dst, CH)])

        pltpu.sync_copy(acc, nsum_o.at[pl.ds(pl.multiple_of(base, CH), BPW)])

    return k(table, neigh_flat, nodes)


def _tc_dense(self_feats, nsum, W1, b1, gamma, beta):
    def body(x_ref, n_ref, w_ref, b_ref, g_ref, bb_ref, o_ref,
             lin_ref, ps_ref):
        p = pl.program_id(0)
        i = pl.program_id(1)

        @pl.when(p == 0)
        def _():
            x = x_ref[...]
            n = n_ref[...] * (1.0 / DEG)
            w = w_ref[...]
            dn = (((1,), (1,)), ((), ()))
            lin = lax.dot_general(x, w[:, :D], dn,
                                  preferred_element_type=jnp.float32)
            lin = lin + lax.dot_general(n, w[:, D:], dn,
                                        preferred_element_type=jnp.float32)
            lin = lin + b_ref[...]
            lin_ref[pl.ds(i * TCB, TCB), :] = lin
            s1 = jnp.sum(lin, axis=0, keepdims=True)
            s2 = jnp.sum(lin * lin, axis=0, keepdims=True)
            ps = jnp.concatenate([s1, s2], axis=0)

            @pl.when(i == 0)
            def _():
                ps_ref[...] = ps

            @pl.when(i > 0)
            def _():
                ps_ref[...] = ps_ref[...] + ps

        @pl.when(p == 1)
        def _():
            ps = ps_ref[...]
            mu = ps[0:1, :] * (1.0 / B)
            var = ps[1:2, :] * (1.0 / B) - mu * mu
            scale = lax.rsqrt(var + 1e-5) * g_ref[...]
            lb = lin_ref[pl.ds(i * TCB, TCB), :]
            o_ref[...] = jnp.maximum((lb - mu) * scale + bb_ref[...], 0.0)

    return pl.pallas_call(
        body,
        grid=(2, NTB),
        in_specs=[
            pl.BlockSpec((TCB, D), lambda p, i: (i * (1 - p), 0)),
            pl.BlockSpec((TCB, D), lambda p, i: (i * (1 - p), 0)),
            pl.BlockSpec((D, 2 * D), lambda p, i: (0, 0)),
            pl.BlockSpec((1, D), lambda p, i: (0, 0)),
            pl.BlockSpec((1, D), lambda p, i: (0, 0)),
            pl.BlockSpec((1, D), lambda p, i: (0, 0)),
        ],
        out_specs=pl.BlockSpec((TCB, D), lambda p, i: (i, 0)),
        out_shape=jax.ShapeDtypeStruct((B, D), jnp.float32),
        scratch_shapes=[
            pltpu.VMEM((B, D), jnp.float32),
            pltpu.VMEM((2, D), jnp.float32),
        ],
    )(self_feats, nsum, W1,
      b1.reshape(1, D), gamma.reshape(1, D), beta.reshape(1, D))


def kernel(nodes, neighbors, emb_table, W1, b1, gamma, beta):
    self_feats, nsum = _sc_gather(emb_table, neighbors.reshape(-1), nodes)
    return _tc_dense(self_feats, nsum, W1, b1, gamma, beta)
